# baseline (speedup 1.0000x reference)
import dataclasses

import jax
import jax.numpy as jnp
from jax import lax
from jax.experimental import pallas as pl
from jax.experimental.pallas import tpu as pltpu
from jax.experimental.pallas import tpu_sc as plsc

M = 4
B = 4096
N = 1600000


_CB = 64000
_GRID_D = N // _CB


def _dense_body(mf_ref, df_ref, fm_ref, sv_ref, ad_ref, sd_ref):
    mm = [mf_ref[:, m, :] for m in range(4)]
    mean3 = (mm[0] + mm[1] + mm[2] + mm[3]) * 0.25
    var3 = sum((x - mean3) * (x - mean3) for x in mm) * (1.0 / 3.0)
    diff3 = mean3 - df_ref[...]
    fm_ref[...] = mean3
    sv_ref[0, :] = jnp.sum(var3, axis=0)
    ad_ref[0, :] = jnp.sum(jnp.abs(diff3), axis=0)
    sd_ref[0, :] = jnp.sum(diff3 * diff3, axis=0)


def _dense_stage(mfw, dfw):
    stat = jax.ShapeDtypeStruct((1, N), jnp.float32)
    return pl.pallas_call(
        _dense_body,
        grid=(_GRID_D,),
        in_specs=[
            pl.BlockSpec((3, 4, _CB), lambda j: (0, 0, j)),
            pl.BlockSpec((3, _CB), lambda j: (0, j)),
        ],
        out_specs=[
            pl.BlockSpec((3, _CB), lambda j: (0, j)),
            pl.BlockSpec((1, _CB), lambda j: (0, j)),
            pl.BlockSpec((1, _CB), lambda j: (0, j)),
            pl.BlockSpec((1, _CB), lambda j: (0, j)),
        ],
        out_shape=[jax.ShapeDtypeStruct((3, N), jnp.float32),
                   stat, stat, stat],
    )(mfw, dfw)


_NW = 32
_PW = N // _NW
_CH = 10000
_NCH = _PW // _CH
_NST = _CH // 16


def _take(x, idx):
    return lax.gather(
        x, idx[:, None],
        dimension_numbers=lax.GatherDimensionNumbers(
            offset_dims=(), collapsed_slice_dims=(0,), start_index_map=(0,)),
        slice_sizes=(1,),
        mode=lax.GatherScatterMode.PROMISE_IN_BOUNDS)


def _sc_body(ids_hbm, sv_hbm, ad_hbm, sd_hbm, out_hbm,
             ids_v0, ids_v1, sv_v0, sv_v1, ad_v0, ad_v1, sd_v0, sd_v1,
             svt, adt, sdt, cntt, maxt, mint, sem0, sem1):
    ids_v = (ids_v0, ids_v1)
    sv_v = (sv_v0, sv_v1)
    ad_v = (ad_v0, ad_v1)
    sd_v = (sd_v0, sd_v1)
    wid = lax.axis_index("s") * 2 + lax.axis_index("c")

    @pl.loop(0, B, step=16)
    def _init(k):
        z = jnp.zeros((16,), jnp.float32)
        svt[pl.ds(k, 16)] = z
        adt[pl.ds(k, 16)] = z
        sdt[pl.ds(k, 16)] = z
        cntt[pl.ds(k, 16)] = z
        maxt[pl.ds(k, 16)] = jnp.full((16,), -jnp.inf, jnp.float32)
        mint[pl.ds(k, 16)] = jnp.full((16,), jnp.inf, jnp.float32)

    iota = lax.iota(jnp.int32, 16)
    ones = jnp.ones((16,), jnp.float32)
    last15 = iota == 15
    nxt = jnp.minimum(iota + 1, 15)
    shifts = [jnp.maximum(iota - d, 0) for d in (1, 2, 4, 8)]
    sems = (sem0, sem1)

    def fire(slot, ci):
        base = wid * _PW + ci * _CH
        sem = sems[slot]
        hs = [pltpu.async_copy(ids_hbm.at[pl.ds(base, _CH)],
                               ids_v[slot], sem),
              pltpu.async_copy(sv_hbm.at[pl.ds(base, _CH)],
                               sv_v[slot], sem),
              pltpu.async_copy(ad_hbm.at[pl.ds(base, _CH)],
                               ad_v[slot], sem),
              pltpu.async_copy(sd_hbm.at[pl.ds(base, _CH)],
                               sd_v[slot], sem)]
        return hs

    def drain(hs):
        for h in hs:
            h.wait()

    def process16(slot, st):
        g = ids_v[slot][pl.ds(st * 16, 16)]
        s_sv = sv_v[slot][pl.ds(st * 16, 16)]
        s_ad = ad_v[slot][pl.ds(st * 16, 16)]
        s_sd = sd_v[slot][pl.ds(st * 16, 16)]
        plsc.addupdate_scatter(svt, [g], s_sv)
        plsc.addupdate_scatter(adt, [g], s_ad)
        plsc.addupdate_scatter(sdt, [g], s_sd)
        plsc.addupdate_scatter(cntt, [g], ones)

        uniform = jnp.max(g) == jnp.min(g)

        @pl.when(uniform)
        def _fast():
            mxs = jnp.max(s_sd)
            mns = jnp.min(s_sd)
            cur_mx = plsc.load_gather(maxt, [g], mask=last15)
            cur_mn = plsc.load_gather(mint, [g], mask=last15)
            plsc.store_scatter(maxt, [g],
                               jnp.maximum(cur_mx, jnp.full((16,), mxs)),
                               mask=last15)
            plsc.store_scatter(mint, [g],
                               jnp.minimum(cur_mn, jnp.full((16,), mns)),
                               mask=last15)

        @pl.when(jnp.logical_not(uniform))
        def _slow():
            mx = s_sd
            mn = s_sd
            for idxd in shifts:
                same = _take(g, idxd) == g
                mx = jnp.where(same, jnp.maximum(mx, _take(mx, idxd)), mx)
                mn = jnp.where(same, jnp.minimum(mn, _take(mn, idxd)), mn)
            lastocc = (g != _take(g, nxt)) | last15
            cur_mx = plsc.load_gather(maxt, [g])
            cur_mn = plsc.load_gather(mint, [g])
            plsc.store_scatter(maxt, [g], jnp.maximum(cur_mx, mx),
                               mask=lastocc)
            plsc.store_scatter(mint, [g], jnp.minimum(cur_mn, mn),
                               mask=lastocc)

    def process(slot):
        @pl.loop(0, _NST - 1, step=2)
        def _pair(st):
            process16(slot, st)
            process16(slot, st + 1)

        process16(slot, _NST - 1)

    hs0 = fire(0, 0)
    drain(hs0)
    @pl.loop(0, (_NCH - 1) // 2)
    def _pipe(j):
        ci = 1 + 2 * j
        h1 = fire(1, ci)
        process(0)
        h0 = fire(0, ci + 1)
        drain(h1)
        process(1)
        drain(h0)

    process(0)

    pltpu.sync_copy(svt, out_hbm.at[0, wid])
    pltpu.sync_copy(adt, out_hbm.at[1, wid])
    pltpu.sync_copy(sdt, out_hbm.at[2, wid])
    pltpu.sync_copy(cntt, out_hbm.at[3, wid])
    pltpu.sync_copy(maxt, out_hbm.at[4, wid])
    pltpu.sync_copy(mint, out_hbm.at[5, wid])


def _segment_stage(image_idx, sv_flat, ad_flat, sd_flat):
    mesh = plsc.VectorSubcoreMesh(core_axis_name="c", subcore_axis_name="s")
    cp = pltpu.CompilerParams()
    if "needs_layout_passes" in pltpu.CompilerParams.__dataclass_fields__:
        cp = dataclasses.replace(cp, needs_layout_passes=False)
    fn = pl.kernel(
        _sc_body,
        out_type=jax.ShapeDtypeStruct((6, _NW, B), jnp.float32),
        mesh=mesh,
        scratch_types=[
            pltpu.VMEM((_CH,), jnp.int32),
            pltpu.VMEM((_CH,), jnp.int32),
            pltpu.VMEM((_CH,), jnp.float32),
            pltpu.VMEM((_CH,), jnp.float32),
            pltpu.VMEM((_CH,), jnp.float32),
            pltpu.VMEM((_CH,), jnp.float32),
            pltpu.VMEM((_CH,), jnp.float32),
            pltpu.VMEM((_CH,), jnp.float32),
            pltpu.VMEM((B,), jnp.float32),
            pltpu.VMEM((B,), jnp.float32),
            pltpu.VMEM((B,), jnp.float32),
            pltpu.VMEM((B,), jnp.float32),
            pltpu.VMEM((B,), jnp.float32),
            pltpu.VMEM((B,), jnp.float32),
            pltpu.SemaphoreType.DMA,
            pltpu.SemaphoreType.DMA,
        ],
        compiler_params=cp,
    )
    return fn(image_idx, sv_flat, ad_flat, sd_flat)


def _combine_body(pt_ref, me_ref, de_ref,
                  emean_o, emax_o, emin_o, evar_o, esd_o, eae_o, ese_o,
                  fvar_o, fsd_o, fae_o, fse_o, fmaxe_o, fmine_o):
    pt = pt_ref[...]
    sv = jnp.sum(pt[0:_NW], axis=0)
    sa = jnp.sum(pt[_NW:2 * _NW], axis=0)
    ss = jnp.sum(pt[2 * _NW:3 * _NW], axis=0)
    cnt = jnp.sum(pt[3 * _NW:4 * _NW], axis=0)
    mx = jnp.max(pt[4 * _NW:5 * _NW], axis=0)
    mn = jnp.min(pt[5 * _NW:6 * _NW], axis=0)
    inv3 = 1.0 / (3.0 * jnp.maximum(cnt, 1.0))
    fvar = sv * inv3
    fvar_o[0, :] = fvar
    fsd_o[0, :] = jnp.sqrt(fvar)
    fae_o[0, :] = sa * inv3
    fse_o[0, :] = ss * inv3
    pos = cnt > 0.0
    fmaxe_o[0, :] = jnp.where(pos, jnp.sqrt(jnp.maximum(mx, 0.0)), -jnp.inf)
    fmine_o[0, :] = jnp.where(pos, jnp.sqrt(jnp.maximum(mn, 0.0)), jnp.inf)

    me = me_ref[...]
    emean = jnp.mean(me, axis=0)
    emean_o[0, :] = emean
    emax_o[...] = jnp.max(me).reshape(1, 1)
    emin_o[...] = jnp.min(me).reshape(1, 1)
    dev = me - emean[None, :]
    evar = jnp.sum(dev * dev, axis=0) * (1.0 / 3.0)
    evar_o[0, :] = evar
    esd_o[0, :] = jnp.sqrt(evar)
    ediff = emean - de_ref[0, :]
    eae_o[0, :] = jnp.abs(ediff)
    ese_o[0, :] = ediff * ediff


def _combine_stage(partials2, me, de2):
    vb = jax.ShapeDtypeStruct((1, B), jnp.float32)
    s1 = jax.ShapeDtypeStruct((1, 1), jnp.float32)
    return pl.pallas_call(
        _combine_body,
        out_shape=[vb, s1, s1, vb, vb, vb, vb, vb, vb, vb, vb, vb, vb],
    )(partials2, me, de2)


def kernel(model_energies, model_forces, data_energy, data_forces, image_idx):
    mfw = model_forces.transpose(2, 0, 1)
    dfw = data_forces.transpose(1, 0)

    fm, sv, ad, sd = _dense_stage(mfw, dfw)

    partials = _segment_stage(image_idx, sv.reshape(N), ad.reshape(N),
                              sd.reshape(N))

    (emean, emax, emin, evar, esd, eae, ese,
     fvar, fsd, fae, fse, fmaxe, fmine) = _combine_stage(
        partials.reshape(6 * _NW, B), model_energies,
        data_energy.reshape(1, B))

    return (emean.reshape(B), fm.transpose(1, 0), emax.reshape(1),
            emin.reshape(1), evar.reshape(B), esd.reshape(B),
            fvar.reshape(B), fsd.reshape(B), eae.reshape(B),
            ese.reshape(B), fae.reshape(B), fse.reshape(B),
            fmaxe.reshape(B), fmine.reshape(B))


# speedup vs baseline: 1.1484x; 1.1484x over previous
import dataclasses

import jax
import jax.numpy as jnp
from jax import lax
from jax.experimental import pallas as pl
from jax.experimental.pallas import tpu as pltpu
from jax.experimental.pallas import tpu_sc as plsc

M = 4
B = 4096
N = 1600000


_CB = 64000
_GRID_D = N // _CB


def _dense_body(mf_ref, df_ref, fm_ref, sv_ref, ad_ref, sd_ref):
    mm = [mf_ref[:, m, :] for m in range(4)]
    mean3 = (mm[0] + mm[1] + mm[2] + mm[3]) * 0.25
    var3 = sum((x - mean3) * (x - mean3) for x in mm) * (1.0 / 3.0)
    diff3 = mean3 - df_ref[...]
    fm_ref[...] = mean3
    sv_ref[0, :] = jnp.sum(var3, axis=0)
    ad_ref[0, :] = jnp.sum(jnp.abs(diff3), axis=0)
    sd_ref[0, :] = jnp.sum(diff3 * diff3, axis=0)


def _dense_stage(mfw, dfw):
    stat = jax.ShapeDtypeStruct((1, N), jnp.float32)
    return pl.pallas_call(
        _dense_body,
        grid=(_GRID_D,),
        in_specs=[
            pl.BlockSpec((3, 4, _CB), lambda j: (0, 0, j)),
            pl.BlockSpec((3, _CB), lambda j: (0, j)),
        ],
        out_specs=[
            pl.BlockSpec((3, _CB), lambda j: (0, j)),
            pl.BlockSpec((1, _CB), lambda j: (0, j)),
            pl.BlockSpec((1, _CB), lambda j: (0, j)),
            pl.BlockSpec((1, _CB), lambda j: (0, j)),
        ],
        out_shape=[jax.ShapeDtypeStruct((3, N), jnp.float32),
                   stat, stat, stat],
    )(mfw, dfw)


_NW = 32
_PW = N // _NW
_CH = 10000
_NCH = _PW // _CH
_NST = _CH // 16


def _take(x, idx):
    return lax.gather(
        x, idx[:, None],
        dimension_numbers=lax.GatherDimensionNumbers(
            offset_dims=(), collapsed_slice_dims=(0,), start_index_map=(0,)),
        slice_sizes=(1,),
        mode=lax.GatherScatterMode.PROMISE_IN_BOUNDS)


def _sc_body(ids_hbm, sv_hbm, ad_hbm, sd_hbm, out_hbm,
             ids_v0, ids_v1, sv_v0, sv_v1, ad_v0, ad_v1, sd_v0, sd_v1,
             svt, adt, sdt, cntt, maxt, mint, sem0, sem1):
    ids_v = (ids_v0, ids_v1)
    sv_v = (sv_v0, sv_v1)
    ad_v = (ad_v0, ad_v1)
    sd_v = (sd_v0, sd_v1)
    wid = lax.axis_index("s") * 2 + lax.axis_index("c")

    @pl.loop(0, B, step=16)
    def _init(k):
        z = jnp.zeros((16,), jnp.float32)
        svt[pl.ds(k, 16)] = z
        adt[pl.ds(k, 16)] = z
        sdt[pl.ds(k, 16)] = z
        cntt[pl.ds(k, 16)] = z
        maxt[pl.ds(k, 16)] = jnp.full((16,), -jnp.inf, jnp.float32)
        mint[pl.ds(k, 16)] = jnp.full((16,), jnp.inf, jnp.float32)

    iota = lax.iota(jnp.int32, 16)
    ones = jnp.ones((16,), jnp.float32)
    last15 = iota == 15
    nxt = jnp.minimum(iota + 1, 15)
    shifts = [jnp.maximum(iota - d, 0) for d in (1, 2, 4, 8)]
    sems = (sem0, sem1)

    def fire(slot, ci):
        base = wid * _PW + ci * _CH
        sem = sems[slot]
        hs = [pltpu.async_copy(ids_hbm.at[pl.ds(base, _CH)],
                               ids_v[slot], sem),
              pltpu.async_copy(sv_hbm.at[pl.ds(base, _CH)],
                               sv_v[slot], sem),
              pltpu.async_copy(ad_hbm.at[pl.ds(base, _CH)],
                               ad_v[slot], sem),
              pltpu.async_copy(sd_hbm.at[pl.ds(base, _CH)],
                               sd_v[slot], sem)]
        return hs

    def drain(hs):
        for h in hs:
            h.wait()

    def _seg16(g, s_sd):
        mx = s_sd
        mn = s_sd
        for idxd in shifts:
            same = _take(g, idxd) == g
            mx = jnp.where(same, jnp.maximum(mx, _take(mx, idxd)), mx)
            mn = jnp.where(same, jnp.minimum(mn, _take(mn, idxd)), mn)
        lastocc = (g != _take(g, nxt)) | last15
        cur_mx = plsc.load_gather(maxt, [g])
        cur_mn = plsc.load_gather(mint, [g])
        plsc.store_scatter(maxt, [g], jnp.maximum(cur_mx, mx), mask=lastocc)
        plsc.store_scatter(mint, [g], jnp.minimum(cur_mn, mn), mask=lastocc)

    def _load_and_sum(slot, st):
        g = ids_v[slot][pl.ds(st * 16, 16)]
        s_sv = sv_v[slot][pl.ds(st * 16, 16)]
        s_ad = ad_v[slot][pl.ds(st * 16, 16)]
        s_sd = sd_v[slot][pl.ds(st * 16, 16)]
        plsc.addupdate_scatter(svt, [g], s_sv)
        plsc.addupdate_scatter(adt, [g], s_ad)
        plsc.addupdate_scatter(sdt, [g], s_sd)
        plsc.addupdate_scatter(cntt, [g], ones)
        return g, s_sd

    def process(slot):
        @pl.loop(0, _NST // 8)
        def _granule(gi):
            gs = []
            ss = []
            for k in range(8):
                g, s_sd = _load_and_sum(slot, gi * 8 + k)
                gs.append(g)
                ss.append(s_sd)
            first = jnp.min(gs[0])
            last = jnp.max(gs[7])

            @pl.when(first == last)
            def _fast():
                m0 = jnp.maximum(jnp.maximum(ss[0], ss[1]),
                                 jnp.maximum(ss[2], ss[3]))
                m1 = jnp.maximum(jnp.maximum(ss[4], ss[5]),
                                 jnp.maximum(ss[6], ss[7]))
                n0 = jnp.minimum(jnp.minimum(ss[0], ss[1]),
                                 jnp.minimum(ss[2], ss[3]))
                n1 = jnp.minimum(jnp.minimum(ss[4], ss[5]),
                                 jnp.minimum(ss[6], ss[7]))
                mxs = jnp.max(jnp.maximum(m0, m1))
                mns = jnp.min(jnp.minimum(n0, n1))
                g7 = gs[7]
                cur_mx = plsc.load_gather(maxt, [g7], mask=last15)
                cur_mn = plsc.load_gather(mint, [g7], mask=last15)
                plsc.store_scatter(maxt, [g7],
                                   jnp.maximum(cur_mx, jnp.full((16,), mxs)),
                                   mask=last15)
                plsc.store_scatter(mint, [g7],
                                   jnp.minimum(cur_mn, jnp.full((16,), mns)),
                                   mask=last15)

            @pl.when(first != last)
            def _slow():
                for k in range(8):
                    _seg16(gs[k], ss[k])

        g, s_sd = _load_and_sum(slot, _NST - 1)
        _seg16(g, s_sd)

    hs0 = fire(0, 0)
    drain(hs0)
    @pl.loop(0, (_NCH - 1) // 2)
    def _pipe(j):
        ci = 1 + 2 * j
        h1 = fire(1, ci)
        process(0)
        h0 = fire(0, ci + 1)
        drain(h1)
        process(1)
        drain(h0)

    process(0)

    pltpu.sync_copy(svt, out_hbm.at[0, wid])
    pltpu.sync_copy(adt, out_hbm.at[1, wid])
    pltpu.sync_copy(sdt, out_hbm.at[2, wid])
    pltpu.sync_copy(cntt, out_hbm.at[3, wid])
    pltpu.sync_copy(maxt, out_hbm.at[4, wid])
    pltpu.sync_copy(mint, out_hbm.at[5, wid])


def _segment_stage(image_idx, sv_flat, ad_flat, sd_flat):
    mesh = plsc.VectorSubcoreMesh(core_axis_name="c", subcore_axis_name="s")
    cp = pltpu.CompilerParams()
    if "needs_layout_passes" in pltpu.CompilerParams.__dataclass_fields__:
        cp = dataclasses.replace(cp, needs_layout_passes=False)
    fn = pl.kernel(
        _sc_body,
        out_type=jax.ShapeDtypeStruct((6, _NW, B), jnp.float32),
        mesh=mesh,
        scratch_types=[
            pltpu.VMEM((_CH,), jnp.int32),
            pltpu.VMEM((_CH,), jnp.int32),
            pltpu.VMEM((_CH,), jnp.float32),
            pltpu.VMEM((_CH,), jnp.float32),
            pltpu.VMEM((_CH,), jnp.float32),
            pltpu.VMEM((_CH,), jnp.float32),
            pltpu.VMEM((_CH,), jnp.float32),
            pltpu.VMEM((_CH,), jnp.float32),
            pltpu.VMEM((B,), jnp.float32),
            pltpu.VMEM((B,), jnp.float32),
            pltpu.VMEM((B,), jnp.float32),
            pltpu.VMEM((B,), jnp.float32),
            pltpu.VMEM((B,), jnp.float32),
            pltpu.VMEM((B,), jnp.float32),
            pltpu.SemaphoreType.DMA,
            pltpu.SemaphoreType.DMA,
        ],
        compiler_params=cp,
    )
    return fn(image_idx, sv_flat, ad_flat, sd_flat)


def _combine_body(pt_ref, me_ref, de_ref,
                  emean_o, emax_o, emin_o, evar_o, esd_o, eae_o, ese_o,
                  fvar_o, fsd_o, fae_o, fse_o, fmaxe_o, fmine_o):
    pt = pt_ref[...]
    sv = jnp.sum(pt[0:_NW], axis=0)
    sa = jnp.sum(pt[_NW:2 * _NW], axis=0)
    ss = jnp.sum(pt[2 * _NW:3 * _NW], axis=0)
    cnt = jnp.sum(pt[3 * _NW:4 * _NW], axis=0)
    mx = jnp.max(pt[4 * _NW:5 * _NW], axis=0)
    mn = jnp.min(pt[5 * _NW:6 * _NW], axis=0)
    inv3 = 1.0 / (3.0 * jnp.maximum(cnt, 1.0))
    fvar = sv * inv3
    fvar_o[0, :] = fvar
    fsd_o[0, :] = jnp.sqrt(fvar)
    fae_o[0, :] = sa * inv3
    fse_o[0, :] = ss * inv3
    pos = cnt > 0.0
    fmaxe_o[0, :] = jnp.where(pos, jnp.sqrt(jnp.maximum(mx, 0.0)), -jnp.inf)
    fmine_o[0, :] = jnp.where(pos, jnp.sqrt(jnp.maximum(mn, 0.0)), jnp.inf)

    me = me_ref[...]
    emean = jnp.mean(me, axis=0)
    emean_o[0, :] = emean
    emax_o[...] = jnp.max(me).reshape(1, 1)
    emin_o[...] = jnp.min(me).reshape(1, 1)
    dev = me - emean[None, :]
    evar = jnp.sum(dev * dev, axis=0) * (1.0 / 3.0)
    evar_o[0, :] = evar
    esd_o[0, :] = jnp.sqrt(evar)
    ediff = emean - de_ref[0, :]
    eae_o[0, :] = jnp.abs(ediff)
    ese_o[0, :] = ediff * ediff


def _combine_stage(partials2, me, de2):
    vb = jax.ShapeDtypeStruct((1, B), jnp.float32)
    s1 = jax.ShapeDtypeStruct((1, 1), jnp.float32)
    return pl.pallas_call(
        _combine_body,
        out_shape=[vb, s1, s1, vb, vb, vb, vb, vb, vb, vb, vb, vb, vb],
    )(partials2, me, de2)


def kernel(model_energies, model_forces, data_energy, data_forces, image_idx):
    mfw = model_forces.transpose(2, 0, 1)
    dfw = data_forces.transpose(1, 0)

    fm, sv, ad, sd = _dense_stage(mfw, dfw)

    partials = _segment_stage(image_idx, sv.reshape(N), ad.reshape(N),
                              sd.reshape(N))

    (emean, emax, emin, evar, esd, eae, ese,
     fvar, fsd, fae, fse, fmaxe, fmine) = _combine_stage(
        partials.reshape(6 * _NW, B), model_energies,
        data_energy.reshape(1, B))

    return (emean.reshape(B), fm.transpose(1, 0), emax.reshape(1),
            emin.reshape(1), evar.reshape(B), esd.reshape(B),
            fvar.reshape(B), fsd.reshape(B), eae.reshape(B),
            ese.reshape(B), fae.reshape(B), fse.reshape(B),
            fmaxe.reshape(B), fmine.reshape(B))


# speedup vs baseline: 1.5964x; 1.3901x over previous
import dataclasses

import jax
import jax.numpy as jnp
from jax import lax
from jax.experimental import pallas as pl
from jax.experimental.pallas import tpu as pltpu
from jax.experimental.pallas import tpu_sc as plsc

M = 4
B = 4096
N = 1600000


_CB = 64000
_GRID_D = N // _CB


def _dense_body(mf_ref, df_ref, fm_ref, sv_ref, ad_ref, sd_ref):
    mm = [mf_ref[:, m, :] for m in range(4)]
    mean3 = (mm[0] + mm[1] + mm[2] + mm[3]) * 0.25
    var3 = sum((x - mean3) * (x - mean3) for x in mm) * (1.0 / 3.0)
    diff3 = mean3 - df_ref[...]
    fm_ref[...] = mean3
    sv_ref[0, :] = jnp.sum(var3, axis=0)
    ad_ref[0, :] = jnp.sum(jnp.abs(diff3), axis=0)
    sd_ref[0, :] = jnp.sum(diff3 * diff3, axis=0)


def _dense_stage(mfw, dfw):
    stat = jax.ShapeDtypeStruct((1, N), jnp.float32)
    return pl.pallas_call(
        _dense_body,
        grid=(_GRID_D,),
        in_specs=[
            pl.BlockSpec((3, 4, _CB), lambda j: (0, 0, j)),
            pl.BlockSpec((3, _CB), lambda j: (0, j)),
        ],
        out_specs=[
            pl.BlockSpec((3, _CB), lambda j: (0, j)),
            pl.BlockSpec((1, _CB), lambda j: (0, j)),
            pl.BlockSpec((1, _CB), lambda j: (0, j)),
            pl.BlockSpec((1, _CB), lambda j: (0, j)),
        ],
        out_shape=[jax.ShapeDtypeStruct((3, N), jnp.float32),
                   stat, stat, stat],
    )(mfw, dfw)


_NW = 32
_PW = N // _NW
_CH = 10000
_NCH = _PW // _CH
_NST = _CH // 16


def _take(x, idx):
    return lax.gather(
        x, idx[:, None],
        dimension_numbers=lax.GatherDimensionNumbers(
            offset_dims=(), collapsed_slice_dims=(0,), start_index_map=(0,)),
        slice_sizes=(1,),
        mode=lax.GatherScatterMode.PROMISE_IN_BOUNDS)


def _sc_body(ids_hbm, sv_hbm, ad_hbm, sd_hbm, out_hbm,
             ids_v0, ids_v1, sv_v0, sv_v1, ad_v0, ad_v1, sd_v0, sd_v1,
             svt, adt, sdt, cntt, maxt, mint, sem0, sem1):
    ids_v = (ids_v0, ids_v1)
    sv_v = (sv_v0, sv_v1)
    ad_v = (ad_v0, ad_v1)
    sd_v = (sd_v0, sd_v1)
    wid = lax.axis_index("s") * 2 + lax.axis_index("c")

    @pl.loop(0, B, step=16)
    def _init(k):
        z = jnp.zeros((16,), jnp.float32)
        svt[pl.ds(k, 16)] = z
        adt[pl.ds(k, 16)] = z
        sdt[pl.ds(k, 16)] = z
        cntt[pl.ds(k, 16)] = z
        maxt[pl.ds(k, 16)] = jnp.full((16,), -jnp.inf, jnp.float32)
        mint[pl.ds(k, 16)] = jnp.full((16,), jnp.inf, jnp.float32)

    iota = lax.iota(jnp.int32, 16)
    ones = jnp.ones((16,), jnp.float32)
    last15 = iota == 15
    nxt = jnp.minimum(iota + 1, 15)
    shifts = [jnp.maximum(iota - d, 0) for d in (1, 2, 4, 8)]
    sems = (sem0, sem1)

    def fire(slot, ci):
        base = wid * _PW + ci * _CH
        sem = sems[slot]
        hs = [pltpu.async_copy(ids_hbm.at[pl.ds(base, _CH)],
                               ids_v[slot], sem),
              pltpu.async_copy(sv_hbm.at[pl.ds(base, _CH)],
                               sv_v[slot], sem),
              pltpu.async_copy(ad_hbm.at[pl.ds(base, _CH)],
                               ad_v[slot], sem),
              pltpu.async_copy(sd_hbm.at[pl.ds(base, _CH)],
                               sd_v[slot], sem)]
        return hs

    def drain(hs):
        for h in hs:
            h.wait()

    def _seg16(g, s_sd):
        mx = s_sd
        mn = s_sd
        for idxd in shifts:
            same = _take(g, idxd) == g
            mx = jnp.where(same, jnp.maximum(mx, _take(mx, idxd)), mx)
            mn = jnp.where(same, jnp.minimum(mn, _take(mn, idxd)), mn)
        lastocc = (g != _take(g, nxt)) | last15
        cur_mx = plsc.load_gather(maxt, [g])
        cur_mn = plsc.load_gather(mint, [g])
        plsc.store_scatter(maxt, [g], jnp.maximum(cur_mx, mx), mask=lastocc)
        plsc.store_scatter(mint, [g], jnp.minimum(cur_mn, mn), mask=lastocc)

    rev8 = [jnp.bitwise_xor(iota, jnp.int32(d)) for d in (1, 2, 4, 8)]

    def _allred(v, op):
        for idxd in rev8:
            v = op(v, _take(v, idxd))
        return v

    def _load16(slot, st):
        g = ids_v[slot][pl.ds(st * 16, 16)]
        s_sv = sv_v[slot][pl.ds(st * 16, 16)]
        s_ad = ad_v[slot][pl.ds(st * 16, 16)]
        s_sd = sd_v[slot][pl.ds(st * 16, 16)]
        return g, s_sv, s_ad, s_sd

    def _scatter_adds(g, s_sv, s_ad, s_sd):
        plsc.addupdate_scatter(svt, [g], s_sv)
        plsc.addupdate_scatter(adt, [g], s_ad)
        plsc.addupdate_scatter(sdt, [g], s_sd)
        plsc.addupdate_scatter(cntt, [g], ones)

    def _tree(vals, op):
        while len(vals) > 1:
            vals = [op(vals[i], vals[i + 1]) for i in range(0, len(vals), 2)]
        return vals[0]

    cnt128 = jnp.full((16,), 128.0, jnp.float32)

    def process(slot):
        @pl.loop(0, _NST // 8)
        def _granule(gi):
            vecs = [_load16(slot, gi * 8 + k) for k in range(8)]
            first = jnp.min(vecs[0][0])
            last = jnp.max(vecs[7][0])

            @pl.when(first == last)
            def _fast():
                g7 = vecs[7][0]
                t_sv = _allred(_tree([v[1] for v in vecs], jnp.add), jnp.add)
                t_ad = _allred(_tree([v[2] for v in vecs], jnp.add), jnp.add)
                sds = [v[3] for v in vecs]
                t_sd = _allred(_tree(sds, jnp.add), jnp.add)
                mxs = _allred(_tree(sds, jnp.maximum), jnp.maximum)
                mns = _allred(_tree(sds, jnp.minimum), jnp.minimum)
                plsc.addupdate_scatter(svt, [g7], t_sv, mask=last15)
                plsc.addupdate_scatter(adt, [g7], t_ad, mask=last15)
                plsc.addupdate_scatter(sdt, [g7], t_sd, mask=last15)
                plsc.addupdate_scatter(cntt, [g7], cnt128, mask=last15)
                cur_mx = plsc.load_gather(maxt, [g7], mask=last15)
                cur_mn = plsc.load_gather(mint, [g7], mask=last15)
                plsc.store_scatter(maxt, [g7], jnp.maximum(cur_mx, mxs),
                                   mask=last15)
                plsc.store_scatter(mint, [g7], jnp.minimum(cur_mn, mns),
                                   mask=last15)

            @pl.when(first != last)
            def _slow():
                for g, s_sv, s_ad, s_sd in vecs:
                    _scatter_adds(g, s_sv, s_ad, s_sd)
                    _seg16(g, s_sd)

        g, s_sv, s_ad, s_sd = _load16(slot, _NST - 1)
        _scatter_adds(g, s_sv, s_ad, s_sd)
        _seg16(g, s_sd)

    hs0 = fire(0, 0)
    drain(hs0)
    @pl.loop(0, (_NCH - 1) // 2)
    def _pipe(j):
        ci = 1 + 2 * j
        h1 = fire(1, ci)
        process(0)
        h0 = fire(0, ci + 1)
        drain(h1)
        process(1)
        drain(h0)

    process(0)

    pltpu.sync_copy(svt, out_hbm.at[0, wid])
    pltpu.sync_copy(adt, out_hbm.at[1, wid])
    pltpu.sync_copy(sdt, out_hbm.at[2, wid])
    pltpu.sync_copy(cntt, out_hbm.at[3, wid])
    pltpu.sync_copy(maxt, out_hbm.at[4, wid])
    pltpu.sync_copy(mint, out_hbm.at[5, wid])


def _segment_stage(image_idx, sv_flat, ad_flat, sd_flat):
    mesh = plsc.VectorSubcoreMesh(core_axis_name="c", subcore_axis_name="s")
    cp = pltpu.CompilerParams()
    if "needs_layout_passes" in pltpu.CompilerParams.__dataclass_fields__:
        cp = dataclasses.replace(cp, needs_layout_passes=False)
    fn = pl.kernel(
        _sc_body,
        out_type=jax.ShapeDtypeStruct((6, _NW, B), jnp.float32),
        mesh=mesh,
        scratch_types=[
            pltpu.VMEM((_CH,), jnp.int32),
            pltpu.VMEM((_CH,), jnp.int32),
            pltpu.VMEM((_CH,), jnp.float32),
            pltpu.VMEM((_CH,), jnp.float32),
            pltpu.VMEM((_CH,), jnp.float32),
            pltpu.VMEM((_CH,), jnp.float32),
            pltpu.VMEM((_CH,), jnp.float32),
            pltpu.VMEM((_CH,), jnp.float32),
            pltpu.VMEM((B,), jnp.float32),
            pltpu.VMEM((B,), jnp.float32),
            pltpu.VMEM((B,), jnp.float32),
            pltpu.VMEM((B,), jnp.float32),
            pltpu.VMEM((B,), jnp.float32),
            pltpu.VMEM((B,), jnp.float32),
            pltpu.SemaphoreType.DMA,
            pltpu.SemaphoreType.DMA,
        ],
        compiler_params=cp,
    )
    return fn(image_idx, sv_flat, ad_flat, sd_flat)


def _combine_body(pt_ref, me_ref, de_ref,
                  emean_o, emax_o, emin_o, evar_o, esd_o, eae_o, ese_o,
                  fvar_o, fsd_o, fae_o, fse_o, fmaxe_o, fmine_o):
    pt = pt_ref[...]
    sv = jnp.sum(pt[0:_NW], axis=0)
    sa = jnp.sum(pt[_NW:2 * _NW], axis=0)
    ss = jnp.sum(pt[2 * _NW:3 * _NW], axis=0)
    cnt = jnp.sum(pt[3 * _NW:4 * _NW], axis=0)
    mx = jnp.max(pt[4 * _NW:5 * _NW], axis=0)
    mn = jnp.min(pt[5 * _NW:6 * _NW], axis=0)
    inv3 = 1.0 / (3.0 * jnp.maximum(cnt, 1.0))
    fvar = sv * inv3
    fvar_o[0, :] = fvar
    fsd_o[0, :] = jnp.sqrt(fvar)
    fae_o[0, :] = sa * inv3
    fse_o[0, :] = ss * inv3
    pos = cnt > 0.0
    fmaxe_o[0, :] = jnp.where(pos, jnp.sqrt(jnp.maximum(mx, 0.0)), -jnp.inf)
    fmine_o[0, :] = jnp.where(pos, jnp.sqrt(jnp.maximum(mn, 0.0)), jnp.inf)

    me = me_ref[...]
    emean = jnp.mean(me, axis=0)
    emean_o[0, :] = emean
    emax_o[...] = jnp.max(me).reshape(1, 1)
    emin_o[...] = jnp.min(me).reshape(1, 1)
    dev = me - emean[None, :]
    evar = jnp.sum(dev * dev, axis=0) * (1.0 / 3.0)
    evar_o[0, :] = evar
    esd_o[0, :] = jnp.sqrt(evar)
    ediff = emean - de_ref[0, :]
    eae_o[0, :] = jnp.abs(ediff)
    ese_o[0, :] = ediff * ediff


def _combine_stage(partials2, me, de2):
    vb = jax.ShapeDtypeStruct((1, B), jnp.float32)
    s1 = jax.ShapeDtypeStruct((1, 1), jnp.float32)
    return pl.pallas_call(
        _combine_body,
        out_shape=[vb, s1, s1, vb, vb, vb, vb, vb, vb, vb, vb, vb, vb],
    )(partials2, me, de2)


def kernel(model_energies, model_forces, data_energy, data_forces, image_idx):
    mfw = model_forces.transpose(2, 0, 1)
    dfw = data_forces.transpose(1, 0)

    fm, sv, ad, sd = _dense_stage(mfw, dfw)

    partials = _segment_stage(image_idx, sv.reshape(N), ad.reshape(N),
                              sd.reshape(N))

    (emean, emax, emin, evar, esd, eae, ese,
     fvar, fsd, fae, fse, fmaxe, fmine) = _combine_stage(
        partials.reshape(6 * _NW, B), model_energies,
        data_energy.reshape(1, B))

    return (emean.reshape(B), fm.transpose(1, 0), emax.reshape(1),
            emin.reshape(1), evar.reshape(B), esd.reshape(B),
            fvar.reshape(B), fsd.reshape(B), eae.reshape(B),
            ese.reshape(B), fae.reshape(B), fse.reshape(B),
            fmaxe.reshape(B), fmine.reshape(B))


# speedup vs baseline: 3.1068x; 1.9461x over previous
import dataclasses

import jax
import jax.numpy as jnp
from jax import lax
from jax.experimental import pallas as pl
from jax.experimental.pallas import tpu as pltpu
from jax.experimental.pallas import tpu_sc as plsc

M = 4
B = 4096
N = 1600000


_CB = 65536
_GRID_D = (N + _CB - 1) // _CB


def _dense_body(mf_ref, df_ref, fm_ref, sv_ref, ad_ref, sd_ref):
    mm = [mf_ref[:, m, :] for m in range(4)]
    mean3 = (mm[0] + mm[1] + mm[2] + mm[3]) * 0.25
    var3 = sum((x - mean3) * (x - mean3) for x in mm) * (1.0 / 3.0)
    diff3 = mean3 - df_ref[...]
    fm_ref[...] = mean3
    sv_ref[...] = jnp.sum(var3, axis=0)
    ad_ref[...] = jnp.sum(jnp.abs(diff3), axis=0)
    sd_ref[...] = jnp.sum(diff3 * diff3, axis=0)


def _dense_stage(mfw, dfw):
    stat = jax.ShapeDtypeStruct((N,), jnp.float32)
    return pl.pallas_call(
        _dense_body,
        grid=(_GRID_D,),
        in_specs=[
            pl.BlockSpec((3, 4, _CB), lambda j: (0, 0, j)),
            pl.BlockSpec((3, _CB), lambda j: (0, j)),
        ],
        out_specs=[
            pl.BlockSpec((3, _CB), lambda j: (0, j)),
            pl.BlockSpec((_CB,), lambda j: (j,)),
            pl.BlockSpec((_CB,), lambda j: (j,)),
            pl.BlockSpec((_CB,), lambda j: (j,)),
        ],
        out_shape=[jax.ShapeDtypeStruct((3, N), jnp.float32),
                   stat, stat, stat],
    )(mfw, dfw)


_NW = 32
_PW = N // _NW
_CH = 10000
_NCH = _PW // _CH
_NST = _CH // 16


def _take(x, idx):
    return lax.gather(
        x, idx[:, None],
        dimension_numbers=lax.GatherDimensionNumbers(
            offset_dims=(), collapsed_slice_dims=(0,), start_index_map=(0,)),
        slice_sizes=(1,),
        mode=lax.GatherScatterMode.PROMISE_IN_BOUNDS)


def _sc_body(ids_hbm, sv_hbm, ad_hbm, sd_hbm, out_hbm,
             ids_v0, ids_v1, sv_v0, sv_v1, ad_v0, ad_v1, sd_v0, sd_v1,
             svt, adt, sdt, cntt, maxt, mint, sem0, sem1):
    ids_v = (ids_v0, ids_v1)
    sv_v = (sv_v0, sv_v1)
    ad_v = (ad_v0, ad_v1)
    sd_v = (sd_v0, sd_v1)
    wid = lax.axis_index("s") * 2 + lax.axis_index("c")

    @pl.loop(0, B, step=16)
    def _init(k):
        z = jnp.zeros((16,), jnp.float32)
        svt[pl.ds(k, 16)] = z
        adt[pl.ds(k, 16)] = z
        sdt[pl.ds(k, 16)] = z
        cntt[pl.ds(k, 16)] = z
        maxt[pl.ds(k, 16)] = jnp.full((16,), -jnp.inf, jnp.float32)
        mint[pl.ds(k, 16)] = jnp.full((16,), jnp.inf, jnp.float32)

    iota = lax.iota(jnp.int32, 16)
    ones = jnp.ones((16,), jnp.float32)
    last15 = iota == 15
    nxt = jnp.minimum(iota + 1, 15)
    shifts = [jnp.maximum(iota - d, 0) for d in (1, 2, 4, 8)]
    sems = (sem0, sem1)

    def fire(slot, ci):
        base = wid * _PW + ci * _CH
        sem = sems[slot]
        hs = [pltpu.async_copy(ids_hbm.at[pl.ds(base, _CH)],
                               ids_v[slot], sem),
              pltpu.async_copy(sv_hbm.at[pl.ds(base, _CH)],
                               sv_v[slot], sem),
              pltpu.async_copy(ad_hbm.at[pl.ds(base, _CH)],
                               ad_v[slot], sem),
              pltpu.async_copy(sd_hbm.at[pl.ds(base, _CH)],
                               sd_v[slot], sem)]
        return hs

    def drain(hs):
        for h in hs:
            h.wait()

    def _seg16(g, s_sd):
        mx = s_sd
        mn = s_sd
        for idxd in shifts:
            same = _take(g, idxd) == g
            mx = jnp.where(same, jnp.maximum(mx, _take(mx, idxd)), mx)
            mn = jnp.where(same, jnp.minimum(mn, _take(mn, idxd)), mn)
        lastocc = (g != _take(g, nxt)) | last15
        cur_mx = plsc.load_gather(maxt, [g])
        cur_mn = plsc.load_gather(mint, [g])
        plsc.store_scatter(maxt, [g], jnp.maximum(cur_mx, mx), mask=lastocc)
        plsc.store_scatter(mint, [g], jnp.minimum(cur_mn, mn), mask=lastocc)

    rev8 = [jnp.bitwise_xor(iota, jnp.int32(d)) for d in (1, 2, 4, 8)]

    def _allred(v, op):
        for idxd in rev8:
            v = op(v, _take(v, idxd))
        return v

    def _load16(slot, st):
        g = ids_v[slot][pl.ds(st * 16, 16)]
        s_sv = sv_v[slot][pl.ds(st * 16, 16)]
        s_ad = ad_v[slot][pl.ds(st * 16, 16)]
        s_sd = sd_v[slot][pl.ds(st * 16, 16)]
        return g, s_sv, s_ad, s_sd

    def _scatter_adds(g, s_sv, s_ad, s_sd):
        plsc.addupdate_scatter(svt, [g], s_sv)
        plsc.addupdate_scatter(adt, [g], s_ad)
        plsc.addupdate_scatter(sdt, [g], s_sd)
        plsc.addupdate_scatter(cntt, [g], ones)

    def _tree(vals, op):
        while len(vals) > 1:
            vals = [op(vals[i], vals[i + 1]) for i in range(0, len(vals), 2)]
        return vals[0]

    cnt128 = jnp.full((16,), 128.0, jnp.float32)

    def process(slot):
        @pl.loop(0, _NST // 8)
        def _granule(gi):
            vecs = [_load16(slot, gi * 8 + k) for k in range(8)]
            first = jnp.min(vecs[0][0])
            last = jnp.max(vecs[7][0])

            @pl.when(first == last)
            def _fast():
                g7 = vecs[7][0]
                t_sv = _allred(_tree([v[1] for v in vecs], jnp.add), jnp.add)
                t_ad = _allred(_tree([v[2] for v in vecs], jnp.add), jnp.add)
                sds = [v[3] for v in vecs]
                t_sd = _allred(_tree(sds, jnp.add), jnp.add)
                mxs = _allred(_tree(sds, jnp.maximum), jnp.maximum)
                mns = _allred(_tree(sds, jnp.minimum), jnp.minimum)
                plsc.addupdate_scatter(svt, [g7], t_sv, mask=last15)
                plsc.addupdate_scatter(adt, [g7], t_ad, mask=last15)
                plsc.addupdate_scatter(sdt, [g7], t_sd, mask=last15)
                plsc.addupdate_scatter(cntt, [g7], cnt128, mask=last15)
                cur_mx = plsc.load_gather(maxt, [g7], mask=last15)
                cur_mn = plsc.load_gather(mint, [g7], mask=last15)
                plsc.store_scatter(maxt, [g7], jnp.maximum(cur_mx, mxs),
                                   mask=last15)
                plsc.store_scatter(mint, [g7], jnp.minimum(cur_mn, mns),
                                   mask=last15)

            @pl.when(first != last)
            def _slow():
                for g, s_sv, s_ad, s_sd in vecs:
                    _scatter_adds(g, s_sv, s_ad, s_sd)
                    _seg16(g, s_sd)

        g, s_sv, s_ad, s_sd = _load16(slot, _NST - 1)
        _scatter_adds(g, s_sv, s_ad, s_sd)
        _seg16(g, s_sd)

    hs0 = fire(0, 0)
    drain(hs0)
    @pl.loop(0, (_NCH - 1) // 2)
    def _pipe(j):
        ci = 1 + 2 * j
        h1 = fire(1, ci)
        process(0)
        h0 = fire(0, ci + 1)
        drain(h1)
        process(1)
        drain(h0)

    process(0)

    pltpu.sync_copy(svt, out_hbm.at[0, wid])
    pltpu.sync_copy(adt, out_hbm.at[1, wid])
    pltpu.sync_copy(sdt, out_hbm.at[2, wid])
    pltpu.sync_copy(cntt, out_hbm.at[3, wid])
    pltpu.sync_copy(maxt, out_hbm.at[4, wid])
    pltpu.sync_copy(mint, out_hbm.at[5, wid])


def _segment_stage(image_idx, sv_flat, ad_flat, sd_flat):
    mesh = plsc.VectorSubcoreMesh(core_axis_name="c", subcore_axis_name="s")
    cp = pltpu.CompilerParams()
    if "needs_layout_passes" in pltpu.CompilerParams.__dataclass_fields__:
        cp = dataclasses.replace(cp, needs_layout_passes=False)
    fn = pl.kernel(
        _sc_body,
        out_type=jax.ShapeDtypeStruct((6, _NW, B), jnp.float32),
        mesh=mesh,
        scratch_types=[
            pltpu.VMEM((_CH,), jnp.int32),
            pltpu.VMEM((_CH,), jnp.int32),
            pltpu.VMEM((_CH,), jnp.float32),
            pltpu.VMEM((_CH,), jnp.float32),
            pltpu.VMEM((_CH,), jnp.float32),
            pltpu.VMEM((_CH,), jnp.float32),
            pltpu.VMEM((_CH,), jnp.float32),
            pltpu.VMEM((_CH,), jnp.float32),
            pltpu.VMEM((B,), jnp.float32),
            pltpu.VMEM((B,), jnp.float32),
            pltpu.VMEM((B,), jnp.float32),
            pltpu.VMEM((B,), jnp.float32),
            pltpu.VMEM((B,), jnp.float32),
            pltpu.VMEM((B,), jnp.float32),
            pltpu.SemaphoreType.DMA,
            pltpu.SemaphoreType.DMA,
        ],
        compiler_params=cp,
    )
    return fn(image_idx, sv_flat, ad_flat, sd_flat)


def _combine_body(pt_ref, me_ref, de_ref,
                  emean_o, emax_o, emin_o, evar_o, esd_o, eae_o, ese_o,
                  fvar_o, fsd_o, fae_o, fse_o, fmaxe_o, fmine_o):
    pt = pt_ref[...]
    sv = jnp.sum(pt[0:_NW], axis=0)
    sa = jnp.sum(pt[_NW:2 * _NW], axis=0)
    ss = jnp.sum(pt[2 * _NW:3 * _NW], axis=0)
    cnt = jnp.sum(pt[3 * _NW:4 * _NW], axis=0)
    mx = jnp.max(pt[4 * _NW:5 * _NW], axis=0)
    mn = jnp.min(pt[5 * _NW:6 * _NW], axis=0)
    inv3 = 1.0 / (3.0 * jnp.maximum(cnt, 1.0))
    fvar = sv * inv3
    fvar_o[0, :] = fvar
    fsd_o[0, :] = jnp.sqrt(fvar)
    fae_o[0, :] = sa * inv3
    fse_o[0, :] = ss * inv3
    pos = cnt > 0.0
    fmaxe_o[0, :] = jnp.where(pos, jnp.sqrt(jnp.maximum(mx, 0.0)), -jnp.inf)
    fmine_o[0, :] = jnp.where(pos, jnp.sqrt(jnp.maximum(mn, 0.0)), jnp.inf)

    me = me_ref[...]
    emean = jnp.mean(me, axis=0)
    emean_o[0, :] = emean
    emax_o[...] = jnp.max(me).reshape(1, 1)
    emin_o[...] = jnp.min(me).reshape(1, 1)
    dev = me - emean[None, :]
    evar = jnp.sum(dev * dev, axis=0) * (1.0 / 3.0)
    evar_o[0, :] = evar
    esd_o[0, :] = jnp.sqrt(evar)
    ediff = emean - de_ref[0, :]
    eae_o[0, :] = jnp.abs(ediff)
    ese_o[0, :] = ediff * ediff


def _combine_stage(partials2, me, de2):
    vb = jax.ShapeDtypeStruct((1, B), jnp.float32)
    s1 = jax.ShapeDtypeStruct((1, 1), jnp.float32)
    return pl.pallas_call(
        _combine_body,
        out_shape=[vb, s1, s1, vb, vb, vb, vb, vb, vb, vb, vb, vb, vb],
    )(partials2, me, de2)


def kernel(model_energies, model_forces, data_energy, data_forces, image_idx):
    mfw = model_forces.transpose(2, 0, 1)
    dfw = data_forces.transpose(1, 0)

    fm, sv, ad, sd = _dense_stage(mfw, dfw)

    partials = _segment_stage(image_idx, sv, ad, sd)

    (emean, emax, emin, evar, esd, eae, ese,
     fvar, fsd, fae, fse, fmaxe, fmine) = _combine_stage(
        partials.reshape(6 * _NW, B), model_energies,
        data_energy.reshape(1, B))

    return (emean.reshape(B), fm.transpose(1, 0), emax.reshape(1),
            emin.reshape(1), evar.reshape(B), esd.reshape(B),
            fvar.reshape(B), fsd.reshape(B), eae.reshape(B),
            ese.reshape(B), fae.reshape(B), fse.reshape(B),
            fmaxe.reshape(B), fmine.reshape(B))
